# baseline (speedup 1.0000x reference)
import functools

import jax
import jax.numpy as jnp
from jax import lax
from jax.experimental import pallas as pl
from jax.experimental.pallas import tpu as pltpu
from jax.experimental.pallas import tpu_sc as plsc

_B = 16384
_V = 100000
_A = 16
_D = 64
_H = 128
_E = 64

_PBLK = 1024
_NPB = (_V + _PBLK - 1) // _PBLK
_PV = _NPB * (_PBLK // 2)

_NC = 2
_NS = 16
_NW = _NC * _NS
_IDS = 2 * _B
_RPW = _IDS // _NW
_CHUNK = 128
_NCHUNK = _RPW // _CHUNK
_HALF = _RPW // 2


def _pack_body(lo_ref, hi_ref, out_ref):
    out_ref[...] = jnp.concatenate([lo_ref[...], hi_ref[...]], axis=1)


def _pack_table(table):
    return pl.pallas_call(
        _pack_body,
        grid=(_NPB,),
        in_specs=[
            pl.BlockSpec((_PBLK // 2, _D), lambda i: (2 * i, 0)),
            pl.BlockSpec((_PBLK // 2, _D), lambda i: (2 * i + 1, 0)),
        ],
        out_specs=pl.BlockSpec((_PBLK // 2, 2 * _D), lambda i: (i, 0)),
        out_shape=jax.ShapeDtypeStruct((_PV, 2 * _D), jnp.float32),
    )(table, table)


def _sc_gather_pairs(ptable, pidx2d):
    mesh = plsc.VectorSubcoreMesh(core_axis_name="c", subcore_axis_name="s")

    @functools.partial(
        pl.kernel,
        mesh=mesh,
        out_type=jax.ShapeDtypeStruct((_IDS, 2 * _D), jnp.float32),
        scratch_types=[
            pltpu.VMEM((_NCHUNK, _CHUNK), jnp.int32),
            pltpu.VMEM((_HALF, 2 * _D), jnp.float32),
            pltpu.SemaphoreType.DMA,
        ],
    )
    def gather_kernel(table_hbm, idx_hbm, out_hbm, idx_v, rows_v, sem):
        wid = lax.axis_index("s") * _NC + lax.axis_index("c")
        pltpu.sync_copy(idx_hbm.at[pl.ds(wid * _NCHUNK, _NCHUNK)], idx_v)
        for h in range(2):
            copies = [
                pltpu.async_copy(
                    table_hbm.at[idx_v.at[h * (_NCHUNK // 2) + j]],
                    rows_v.at[pl.ds(j * _CHUNK, _CHUNK)],
                    sem,
                )
                for j in range(_NCHUNK // 2)
            ]
            for c in copies:
                c.wait()
            pltpu.sync_copy(
                rows_v, out_hbm.at[pl.ds(wid * _RPW + h * _HALF, _HALF)])

    return gather_kernel(ptable, pidx2d)


_BLK = 1024
_NB = _B // _BLK


def _mlp_body(gs_ref, gn_ref, hs_ref, hn_ref, aid_ref, at_ref,
              w1ss_ref, w1ns_ref, w1a_ref, b1_ref, w2_ref, b2_ref, out_ref):
    halfidx = lax.broadcasted_iota(jnp.int32, (_BLK, 2 * _D), 1) // _D
    ms = (hs_ref[0, 0, :][:, None] == halfidx).astype(jnp.float32)
    mn = (hn_ref[0, 0, :][:, None] == halfidx).astype(jnp.float32)
    se = gs_ref[...] * ms
    ne = gn_ref[...] * mn
    aid = aid_ref[0, 0, :]
    onehot = (aid[:, None] == lax.broadcasted_iota(jnp.int32, (_BLK, _A), 1)
              ).astype(jnp.float32)
    aw = jnp.dot(at_ref[...], w1a_ref[...], preferred_element_type=jnp.float32)
    acc = jnp.dot(se, w1ss_ref[...], preferred_element_type=jnp.float32)
    acc = acc + jnp.dot(ne, w1ns_ref[...], preferred_element_type=jnp.float32)
    acc = acc + jnp.dot(onehot, aw, preferred_element_type=jnp.float32)
    h = jnp.maximum(acc + b1_ref[...], 0.0)
    out_ref[...] = jnp.dot(h, w2_ref[...], preferred_element_type=jnp.float32) + b2_ref[...]


def _mlp(gathered, hs3, hn3, aid3, action_table, w1ss, w1ns, w1a, b1r, W2, b2r):
    return pl.pallas_call(
        _mlp_body,
        grid=(_NB,),
        in_specs=[
            pl.BlockSpec((_BLK, 2 * _D), lambda i: (i, 0)),
            pl.BlockSpec((_BLK, 2 * _D), lambda i: (i + _NB, 0)),
            pl.BlockSpec((1, 1, _BLK), lambda i: (i, 0, 0)),
            pl.BlockSpec((1, 1, _BLK), lambda i: (i, 0, 0)),
            pl.BlockSpec((1, 1, _BLK), lambda i: (i, 0, 0)),
            pl.BlockSpec((_A, _D), lambda i: (0, 0)),
            pl.BlockSpec((2 * _D, _H), lambda i: (0, 0)),
            pl.BlockSpec((2 * _D, _H), lambda i: (0, 0)),
            pl.BlockSpec((_D, _H), lambda i: (0, 0)),
            pl.BlockSpec((1, _H), lambda i: (0, 0)),
            pl.BlockSpec((_H, _E), lambda i: (0, 0)),
            pl.BlockSpec((1, _E), lambda i: (0, 0)),
        ],
        out_specs=pl.BlockSpec((_BLK, _E), lambda i: (i, 0)),
        out_shape=jax.ShapeDtypeStruct((_B, _E), jnp.float32),
    )(gathered, gathered, hs3, hn3, aid3, action_table,
      w1ss, w1ns, w1a, b1r, W2, b2r)


def kernel(state_ids, next_state_ids, action_ids, state_table, action_table,
           W1, b1, W2, b2):
    sid = state_ids.astype(jnp.int32)
    nid = next_state_ids.astype(jnp.int32)
    ids = jnp.concatenate([sid, nid])
    prow = (ids >> 10) * (_PBLK // 2) + (ids & (_PBLK // 2 - 1))
    pidx2d = prow.reshape(_IDS // _CHUNK, _CHUNK)
    ptable = _pack_table(state_table)
    gathered = _sc_gather_pairs(ptable, pidx2d)
    hs3 = ((sid >> 9) & 1).reshape(_NB, 1, _BLK)
    hn3 = ((nid >> 9) & 1).reshape(_NB, 1, _BLK)
    aid3 = action_ids.astype(jnp.int32).reshape(_NB, 1, _BLK)
    w1ss = jnp.concatenate([W1[:_D], W1[:_D]], axis=0)
    w1ns = jnp.concatenate([W1[_D:2 * _D], W1[_D:2 * _D]], axis=0)
    return _mlp(gathered, hs3, hn3, aid3, action_table, w1ss, w1ns,
                W1[2 * _D:], b1.reshape(1, _H), W2, b2.reshape(1, _E))


# speedup vs baseline: 1.2695x; 1.2695x over previous
import functools

import jax
import jax.numpy as jnp
from jax import lax
from jax.experimental import pallas as pl
from jax.experimental.pallas import tpu as pltpu
from jax.experimental.pallas import tpu_sc as plsc

_B = 16384
_V = 100000
_A = 16
_D = 64
_H = 128
_E = 64

_NC = 2
_NS = 16
_NW = _NC * _NS
_IDS = 2 * _B
_RPW = _IDS // _NW
_CHUNK = 128
_NCHUNK = _RPW // _CHUNK
_HALF = _RPW // 2


_PCOLS = 1024
_NPB = (_V + _PCOLS - 1) // _PCOLS
_PV = _NPB * (_PCOLS // 2)


def _pack_body(tt_ref, out_ref):
    t = tt_ref[...].T
    out_ref[...] = jnp.concatenate(
        [t[:_PCOLS // 2], t[_PCOLS // 2:]], axis=1)


def _pack_table(tt):
    return pl.pallas_call(
        _pack_body,
        grid=(_NPB,),
        in_specs=[pl.BlockSpec((_D, _PCOLS), lambda i: (0, i))],
        out_specs=pl.BlockSpec((_PCOLS // 2, 2 * _D), lambda i: (i, 0)),
        out_shape=jax.ShapeDtypeStruct((_PV, 2 * _D), jnp.float32),
    )(tt)


def _sc_gather_pairs(table2, pidx2d):
    mesh = plsc.VectorSubcoreMesh(core_axis_name="c", subcore_axis_name="s")

    @functools.partial(
        pl.kernel,
        mesh=mesh,
        out_type=jax.ShapeDtypeStruct((_IDS, 2 * _D), jnp.float32),
        scratch_types=[
            pltpu.VMEM((_NCHUNK, _CHUNK), jnp.int32),
            pltpu.VMEM((_HALF, 2 * _D), jnp.float32),
            pltpu.SemaphoreType.DMA,
        ],
    )
    def gather_kernel(table_hbm, idx_hbm, out_hbm, idx_v, rows_v, sem):
        wid = lax.axis_index("s") * _NC + lax.axis_index("c")
        pltpu.sync_copy(idx_hbm.at[pl.ds(wid * _NCHUNK, _NCHUNK)], idx_v)
        for h in range(2):
            copies = [
                pltpu.async_copy(
                    table_hbm.at[idx_v.at[h * (_NCHUNK // 2) + j]],
                    rows_v.at[pl.ds(j * _CHUNK, _CHUNK)],
                    sem,
                )
                for j in range(_NCHUNK // 2)
            ]
            for c in copies:
                c.wait()
            pltpu.sync_copy(
                rows_v, out_hbm.at[pl.ds(wid * _RPW + h * _HALF, _HALF)])

    return gather_kernel(table2, pidx2d)


_BLK = 1024
_NB = _B // _BLK


def _mlp_body(gs_ref, gn_ref, hs_ref, hn_ref, aid_ref, at_ref,
              w1ss_ref, w1ns_ref, w1a_ref, b1_ref, w2_ref, b2_ref, out_ref):
    halfidx = lax.broadcasted_iota(jnp.int32, (_BLK, 2 * _D), 1) // _D
    ms = (hs_ref[0, 0, :][:, None] == halfidx).astype(jnp.float32)
    mn = (hn_ref[0, 0, :][:, None] == halfidx).astype(jnp.float32)
    se = gs_ref[...] * ms
    ne = gn_ref[...] * mn
    aid = aid_ref[0, 0, :]
    onehot = (aid[:, None] == lax.broadcasted_iota(jnp.int32, (_BLK, _A), 1)
              ).astype(jnp.float32)
    aw = jnp.dot(at_ref[...], w1a_ref[...], preferred_element_type=jnp.float32)
    acc = jnp.dot(se, w1ss_ref[...], preferred_element_type=jnp.float32)
    acc = acc + jnp.dot(ne, w1ns_ref[...], preferred_element_type=jnp.float32)
    acc = acc + jnp.dot(onehot, aw, preferred_element_type=jnp.float32)
    h = jnp.maximum(acc + b1_ref[...], 0.0)
    out_ref[...] = jnp.dot(h, w2_ref[...], preferred_element_type=jnp.float32) + b2_ref[...]


def _mlp(gathered, hs3, hn3, aid3, action_table, w1ss, w1ns, w1a, b1r, W2, b2r):
    return pl.pallas_call(
        _mlp_body,
        grid=(_NB,),
        in_specs=[
            pl.BlockSpec((_BLK, 2 * _D), lambda i: (i, 0)),
            pl.BlockSpec((_BLK, 2 * _D), lambda i: (i + _NB, 0)),
            pl.BlockSpec((1, 1, _BLK), lambda i: (i, 0, 0)),
            pl.BlockSpec((1, 1, _BLK), lambda i: (i, 0, 0)),
            pl.BlockSpec((1, 1, _BLK), lambda i: (i, 0, 0)),
            pl.BlockSpec((_A, _D), lambda i: (0, 0)),
            pl.BlockSpec((2 * _D, _H), lambda i: (0, 0)),
            pl.BlockSpec((2 * _D, _H), lambda i: (0, 0)),
            pl.BlockSpec((_D, _H), lambda i: (0, 0)),
            pl.BlockSpec((1, _H), lambda i: (0, 0)),
            pl.BlockSpec((_H, _E), lambda i: (0, 0)),
            pl.BlockSpec((1, _E), lambda i: (0, 0)),
        ],
        out_specs=pl.BlockSpec((_BLK, _E), lambda i: (i, 0)),
        out_shape=jax.ShapeDtypeStruct((_B, _E), jnp.float32),
    )(gathered, gathered, hs3, hn3, aid3, action_table,
      w1ss, w1ns, w1a, b1r, W2, b2r)


def kernel(state_ids, next_state_ids, action_ids, state_table, action_table,
           W1, b1, W2, b2):
    sid = state_ids.astype(jnp.int32)
    nid = next_state_ids.astype(jnp.int32)
    ids = jnp.concatenate([sid, nid])
    pidx2d = ((ids >> 10) * (_PCOLS // 2) + (ids & (_PCOLS // 2 - 1))
              ).reshape(_IDS // _CHUNK, _CHUNK)
    table2 = _pack_table(state_table.T)
    gathered = _sc_gather_pairs(table2, pidx2d)
    hs3 = ((sid >> 9) & 1).reshape(_NB, 1, _BLK)
    hn3 = ((nid >> 9) & 1).reshape(_NB, 1, _BLK)
    aid3 = action_ids.astype(jnp.int32).reshape(_NB, 1, _BLK)
    w1ss = jnp.concatenate([W1[:_D], W1[:_D]], axis=0)
    w1ns = jnp.concatenate([W1[_D:2 * _D], W1[_D:2 * _D]], axis=0)
    return _mlp(gathered, hs3, hn3, aid3, action_table, w1ss, w1ns,
                W1[2 * _D:], b1.reshape(1, _H), W2, b2.reshape(1, _E))


# speedup vs baseline: 1.8706x; 1.4735x over previous
import functools

import jax
import jax.numpy as jnp
from jax import lax
from jax.experimental import pallas as pl
from jax.experimental.pallas import tpu as pltpu
from jax.experimental.pallas import tpu_sc as plsc

_B = 16384
_V = 100000
_A = 16
_D = 64
_H = 128
_E = 64

_NC = 2
_NS = 16
_NW = _NC * _NS
_IDS = 2 * _B
_RPW = _IDS // _NW
_CHUNK = 128
_NCHUNK = _RPW // _CHUNK
_HALF = _RPW // 2


_PCOLS = 8192
_PSH = 13
_NPB = (_V + _PCOLS - 1) // _PCOLS
_PV = _NPB * (_PCOLS // 2)


def _pack_body(tt_ref, out_ref):
    t = tt_ref[...].T
    out_ref[...] = jnp.concatenate(
        [t[:_PCOLS // 2], t[_PCOLS // 2:]], axis=1)


def _pack_table(tt):
    return pl.pallas_call(
        _pack_body,
        grid=(_NPB,),
        in_specs=[pl.BlockSpec((_D, _PCOLS), lambda i: (0, i))],
        out_specs=pl.BlockSpec((_PCOLS // 2, 2 * _D), lambda i: (i, 0)),
        out_shape=jax.ShapeDtypeStruct((_PV, 2 * _D), jnp.float32),
    )(tt)


def _sc_gather_pairs(table2, pidx2d):
    mesh = plsc.VectorSubcoreMesh(core_axis_name="c", subcore_axis_name="s")

    @functools.partial(
        pl.kernel,
        mesh=mesh,
        out_type=jax.ShapeDtypeStruct((_IDS, 2 * _D), jnp.float32),
        scratch_types=[
            pltpu.VMEM((_NCHUNK, _CHUNK), jnp.int32),
            pltpu.VMEM((_HALF, 2 * _D), jnp.float32),
            pltpu.SemaphoreType.DMA,
        ],
    )
    def gather_kernel(table_hbm, idx_hbm, out_hbm, idx_v, rows_v, sem):
        wid = lax.axis_index("s") * _NC + lax.axis_index("c")
        pltpu.sync_copy(idx_hbm.at[pl.ds(wid * _NCHUNK, _NCHUNK)], idx_v)
        for h in range(2):
            copies = [
                pltpu.async_copy(
                    table_hbm.at[idx_v.at[h * (_NCHUNK // 2) + j]],
                    rows_v.at[pl.ds(j * _CHUNK, _CHUNK)],
                    sem,
                )
                for j in range(_NCHUNK // 2)
            ]
            for c in copies:
                c.wait()
            pltpu.sync_copy(
                rows_v, out_hbm.at[pl.ds(wid * _RPW + h * _HALF, _HALF)])

    return gather_kernel(table2, pidx2d)


_BLK = 1024
_NB = _B // _BLK


def _mlp_body(gs_ref, gn_ref, hs_ref, hn_ref, aid_ref, at_ref,
              w1ss_ref, w1ns_ref, w1a_ref, b1_ref, w2_ref, b2_ref, out_ref):
    halfidx = lax.broadcasted_iota(jnp.int32, (_BLK, 2 * _D), 1) // _D
    ms = (hs_ref[0, 0, :][:, None] == halfidx).astype(jnp.float32)
    mn = (hn_ref[0, 0, :][:, None] == halfidx).astype(jnp.float32)
    se = gs_ref[...] * ms
    ne = gn_ref[...] * mn
    aid = aid_ref[0, 0, :]
    onehot = (aid[:, None] == lax.broadcasted_iota(jnp.int32, (_BLK, _A), 1)
              ).astype(jnp.float32)
    aw = jnp.dot(at_ref[...], w1a_ref[...], preferred_element_type=jnp.float32)
    acc = jnp.dot(se, w1ss_ref[...], preferred_element_type=jnp.float32)
    acc = acc + jnp.dot(ne, w1ns_ref[...], preferred_element_type=jnp.float32)
    acc = acc + jnp.dot(onehot, aw, preferred_element_type=jnp.float32)
    h = jnp.maximum(acc + b1_ref[...], 0.0)
    out_ref[...] = jnp.dot(h, w2_ref[...], preferred_element_type=jnp.float32) + b2_ref[...]


def _mlp(gathered, hs3, hn3, aid3, action_table, w1ss, w1ns, w1a, b1r, W2, b2r):
    return pl.pallas_call(
        _mlp_body,
        grid=(_NB,),
        in_specs=[
            pl.BlockSpec((_BLK, 2 * _D), lambda i: (i, 0)),
            pl.BlockSpec((_BLK, 2 * _D), lambda i: (i + _NB, 0)),
            pl.BlockSpec((1, 1, _BLK), lambda i: (i, 0, 0)),
            pl.BlockSpec((1, 1, _BLK), lambda i: (i, 0, 0)),
            pl.BlockSpec((1, 1, _BLK), lambda i: (i, 0, 0)),
            pl.BlockSpec((_A, _D), lambda i: (0, 0)),
            pl.BlockSpec((2 * _D, _H), lambda i: (0, 0)),
            pl.BlockSpec((2 * _D, _H), lambda i: (0, 0)),
            pl.BlockSpec((_D, _H), lambda i: (0, 0)),
            pl.BlockSpec((1, _H), lambda i: (0, 0)),
            pl.BlockSpec((_H, _E), lambda i: (0, 0)),
            pl.BlockSpec((1, _E), lambda i: (0, 0)),
        ],
        out_specs=pl.BlockSpec((_BLK, _E), lambda i: (i, 0)),
        out_shape=jax.ShapeDtypeStruct((_B, _E), jnp.float32),
    )(gathered, gathered, hs3, hn3, aid3, action_table,
      w1ss, w1ns, w1a, b1r, W2, b2r)


def kernel(state_ids, next_state_ids, action_ids, state_table, action_table,
           W1, b1, W2, b2):
    sid = state_ids.astype(jnp.int32)
    nid = next_state_ids.astype(jnp.int32)
    ids = jnp.concatenate([sid, nid])
    pidx2d = ((ids >> _PSH) * (_PCOLS // 2) + (ids & (_PCOLS // 2 - 1))
              ).reshape(_IDS // _CHUNK, _CHUNK)
    table2 = _pack_table(state_table.T)
    gathered = _sc_gather_pairs(table2, pidx2d)
    hs3 = ((sid >> (_PSH - 1)) & 1).reshape(_NB, 1, _BLK)
    hn3 = ((nid >> (_PSH - 1)) & 1).reshape(_NB, 1, _BLK)
    aid3 = action_ids.astype(jnp.int32).reshape(_NB, 1, _BLK)
    w1ss = jnp.concatenate([W1[:_D], W1[:_D]], axis=0)
    w1ns = jnp.concatenate([W1[_D:2 * _D], W1[_D:2 * _D]], axis=0)
    return _mlp(gathered, hs3, hn3, aid3, action_table, w1ss, w1ns,
                W1[2 * _D:], b1.reshape(1, _H), W2, b2.reshape(1, _E))


# speedup vs baseline: 1.8880x; 1.0093x over previous
import functools

import jax
import jax.numpy as jnp
from jax import lax
from jax.experimental import pallas as pl
from jax.experimental.pallas import tpu as pltpu
from jax.experimental.pallas import tpu_sc as plsc

_B = 16384
_V = 100000
_A = 16
_D = 64
_H = 128
_E = 64

_NC = 2
_NS = 16
_NW = _NC * _NS
_IDS = 2 * _B
_RPW = _IDS // _NW
_CHUNK = 128
_NCHUNK = _RPW // _CHUNK
_HALF = _RPW // 2


_PCOLS = 16384
_PSH = 14
_NPB = (_V + _PCOLS - 1) // _PCOLS
_PV = _NPB * (_PCOLS // 2)


def _pack_body(tt_ref, out_ref):
    t = tt_ref[...].T
    out_ref[...] = jnp.concatenate(
        [t[:_PCOLS // 2], t[_PCOLS // 2:]], axis=1)


def _pack_table(tt):
    return pl.pallas_call(
        _pack_body,
        grid=(_NPB,),
        in_specs=[pl.BlockSpec((_D, _PCOLS), lambda i: (0, i))],
        out_specs=pl.BlockSpec((_PCOLS // 2, 2 * _D), lambda i: (i, 0)),
        out_shape=jax.ShapeDtypeStruct((_PV, 2 * _D), jnp.float32),
    )(tt)


def _sc_gather_pairs(table2, pidx2d):
    mesh = plsc.VectorSubcoreMesh(core_axis_name="c", subcore_axis_name="s")

    @functools.partial(
        pl.kernel,
        mesh=mesh,
        out_type=jax.ShapeDtypeStruct((_IDS, 2 * _D), jnp.float32),
        scratch_types=[
            pltpu.VMEM((_NCHUNK, _CHUNK), jnp.int32),
            pltpu.VMEM((_HALF, 2 * _D), jnp.float32),
            pltpu.SemaphoreType.DMA,
        ],
    )
    def gather_kernel(table_hbm, idx_hbm, out_hbm, idx_v, rows_v, sem):
        wid = lax.axis_index("s") * _NC + lax.axis_index("c")
        pltpu.sync_copy(idx_hbm.at[pl.ds(wid * _NCHUNK, _NCHUNK)], idx_v)
        for h in range(2):
            copies = [
                pltpu.async_copy(
                    table_hbm.at[idx_v.at[h * (_NCHUNK // 2) + j]],
                    rows_v.at[pl.ds(j * _CHUNK, _CHUNK)],
                    sem,
                )
                for j in range(_NCHUNK // 2)
            ]
            for c in copies:
                c.wait()
            pltpu.sync_copy(
                rows_v, out_hbm.at[pl.ds(wid * _RPW + h * _HALF, _HALF)])

    return gather_kernel(table2, pidx2d)


_BLK = 1024
_NB = _B // _BLK


def _mlp_body(gs_ref, gn_ref, hs_ref, hn_ref, aid_ref, at_ref,
              w1ss_ref, w1ns_ref, w1a_ref, b1_ref, w2_ref, b2_ref, out_ref):
    halfidx = lax.broadcasted_iota(jnp.int32, (_BLK, 2 * _D), 1) // _D
    ms = (hs_ref[0, 0, :][:, None] == halfidx).astype(jnp.float32)
    mn = (hn_ref[0, 0, :][:, None] == halfidx).astype(jnp.float32)
    se = gs_ref[...] * ms
    ne = gn_ref[...] * mn
    aid = aid_ref[0, 0, :]
    onehot = (aid[:, None] == lax.broadcasted_iota(jnp.int32, (_BLK, _A), 1)
              ).astype(jnp.float32)
    aw = jnp.dot(at_ref[...], w1a_ref[...], preferred_element_type=jnp.float32)
    acc = jnp.dot(se, w1ss_ref[...], preferred_element_type=jnp.float32)
    acc = acc + jnp.dot(ne, w1ns_ref[...], preferred_element_type=jnp.float32)
    acc = acc + jnp.dot(onehot, aw, preferred_element_type=jnp.float32)
    h = jnp.maximum(acc + b1_ref[...], 0.0)
    out_ref[...] = jnp.dot(h, w2_ref[...], preferred_element_type=jnp.float32) + b2_ref[...]


def _mlp(gathered, hs3, hn3, aid3, action_table, w1ss, w1ns, w1a, b1r, W2, b2r):
    return pl.pallas_call(
        _mlp_body,
        grid=(_NB,),
        in_specs=[
            pl.BlockSpec((_BLK, 2 * _D), lambda i: (i, 0)),
            pl.BlockSpec((_BLK, 2 * _D), lambda i: (i + _NB, 0)),
            pl.BlockSpec((1, 1, _BLK), lambda i: (i, 0, 0)),
            pl.BlockSpec((1, 1, _BLK), lambda i: (i, 0, 0)),
            pl.BlockSpec((1, 1, _BLK), lambda i: (i, 0, 0)),
            pl.BlockSpec((_A, _D), lambda i: (0, 0)),
            pl.BlockSpec((2 * _D, _H), lambda i: (0, 0)),
            pl.BlockSpec((2 * _D, _H), lambda i: (0, 0)),
            pl.BlockSpec((_D, _H), lambda i: (0, 0)),
            pl.BlockSpec((1, _H), lambda i: (0, 0)),
            pl.BlockSpec((_H, _E), lambda i: (0, 0)),
            pl.BlockSpec((1, _E), lambda i: (0, 0)),
        ],
        out_specs=pl.BlockSpec((_BLK, _E), lambda i: (i, 0)),
        out_shape=jax.ShapeDtypeStruct((_B, _E), jnp.float32),
    )(gathered, gathered, hs3, hn3, aid3, action_table,
      w1ss, w1ns, w1a, b1r, W2, b2r)


def kernel(state_ids, next_state_ids, action_ids, state_table, action_table,
           W1, b1, W2, b2):
    sid = state_ids.astype(jnp.int32)
    nid = next_state_ids.astype(jnp.int32)
    ids = jnp.concatenate([sid, nid])
    pidx2d = ((ids >> _PSH) * (_PCOLS // 2) + (ids & (_PCOLS // 2 - 1))
              ).reshape(_IDS // _CHUNK, _CHUNK)
    table2 = _pack_table(state_table.T)
    gathered = _sc_gather_pairs(table2, pidx2d)
    hs3 = ((sid >> (_PSH - 1)) & 1).reshape(_NB, 1, _BLK)
    hn3 = ((nid >> (_PSH - 1)) & 1).reshape(_NB, 1, _BLK)
    aid3 = action_ids.astype(jnp.int32).reshape(_NB, 1, _BLK)
    w1ss = jnp.concatenate([W1[:_D], W1[:_D]], axis=0)
    w1ns = jnp.concatenate([W1[_D:2 * _D], W1[_D:2 * _D]], axis=0)
    return _mlp(gathered, hs3, hn3, aid3, action_table, w1ss, w1ns,
                W1[2 * _D:], b1.reshape(1, _H), W2, b2.reshape(1, _E))


# speedup vs baseline: 2.1057x; 1.1153x over previous
import functools

import jax
import jax.numpy as jnp
from jax import lax
from jax.experimental import pallas as pl
from jax.experimental.pallas import tpu as pltpu
from jax.experimental.pallas import tpu_sc as plsc

_B = 16384
_V = 100000
_A = 16
_D = 64
_H = 128
_E = 64

_PCOLS = 16384
_PSH = 14
_NPB = (_V + _PCOLS - 1) // _PCOLS
_PV = _NPB * (_PCOLS // 2)

_NC = 2
_NS = 16
_NW = _NC * _NS
_RPW = _B // _NW
_CHUNK = 128
_NCHUNK = _RPW // _CHUNK


def _pack_body(tt_ref, out_ref):
    t = tt_ref[...].T
    out_ref[...] = jnp.concatenate(
        [t[:_PCOLS // 2], t[_PCOLS // 2:]], axis=1)


def _pack_table(tt):
    return pl.pallas_call(
        _pack_body,
        grid=(_NPB,),
        in_specs=[pl.BlockSpec((_D, _PCOLS), lambda i: (0, i))],
        out_specs=pl.BlockSpec((_PCOLS // 2, 2 * _D), lambda i: (i, 0)),
        out_shape=jax.ShapeDtypeStruct((_PV, 2 * _D), jnp.float32),
    )(tt)


def _sc_gather_packed(table, sid2d, nid2d):
    mesh = plsc.VectorSubcoreMesh(core_axis_name="c", subcore_axis_name="s")

    @functools.partial(
        pl.kernel,
        mesh=mesh,
        out_type=jax.ShapeDtypeStruct((_B, 2 * _D), jnp.float32),
        scratch_types=[
            pltpu.VMEM((_NCHUNK, _CHUNK), jnp.int32),
            pltpu.VMEM((_NCHUNK, _CHUNK), jnp.int32),
            pltpu.VMEM((_RPW, _D), jnp.float32),
            pltpu.VMEM((_RPW, _D), jnp.float32),
            pltpu.SemaphoreType.DMA,
        ],
        compiler_params=pltpu.CompilerParams(use_tc_tiling_on_sc=False),
    )
    def gather_kernel(table_hbm, sid_hbm, nid_hbm, out_hbm, sidx_v, nidx_v,
                      srows_v, nrows_v, sem):
        wid = lax.axis_index("s") * _NC + lax.axis_index("c")
        pltpu.sync_copy(sid_hbm.at[pl.ds(wid * _NCHUNK, _NCHUNK)], sidx_v)
        pltpu.sync_copy(nid_hbm.at[pl.ds(wid * _NCHUNK, _NCHUNK)], nidx_v)
        copies = []
        for j in range(_NCHUNK):
            copies.append(pltpu.async_copy(
                table_hbm.at[sidx_v.at[j]],
                srows_v.at[pl.ds(j * _CHUNK, _CHUNK)],
                sem,
            ))
            copies.append(pltpu.async_copy(
                table_hbm.at[nidx_v.at[j]],
                nrows_v.at[pl.ds(j * _CHUNK, _CHUNK)],
                sem,
            ))
        for c in copies:
            c.wait()
        pltpu.sync_copy(
            srows_v, out_hbm.at[pl.ds(wid * _RPW, _RPW), pl.ds(0, _D)])
        pltpu.sync_copy(
            nrows_v, out_hbm.at[pl.ds(wid * _RPW, _RPW), pl.ds(_D, _D)])

    return gather_kernel(table, sid2d, nid2d)


_BLK = 1024
_NB = _B // _BLK


def _mlp_body(g_ref, aid_ref, at_ref, w1sn_ref, w1a_ref, b1_ref, w2_ref,
              b2_ref, out_ref):
    aid = aid_ref[0, 0, :]
    onehot = (aid[:, None] == lax.broadcasted_iota(jnp.int32, (_BLK, _A), 1)
              ).astype(jnp.float32)
    aw = jnp.dot(at_ref[...], w1a_ref[...], preferred_element_type=jnp.float32)
    acc = jnp.dot(g_ref[...], w1sn_ref[...], preferred_element_type=jnp.float32)
    acc = acc + jnp.dot(onehot, aw, preferred_element_type=jnp.float32)
    h = jnp.maximum(acc + b1_ref[...], 0.0)
    out_ref[...] = jnp.dot(h, w2_ref[...], preferred_element_type=jnp.float32) + b2_ref[...]


def _mlp(gathered, aid3, action_table, w1sn, w1a, b1r, W2, b2r):
    return pl.pallas_call(
        _mlp_body,
        grid=(_NB,),
        in_specs=[
            pl.BlockSpec((_BLK, 2 * _D), lambda i: (i, 0)),
            pl.BlockSpec((1, 1, _BLK), lambda i: (i, 0, 0)),
            pl.BlockSpec((_A, _D), lambda i: (0, 0)),
            pl.BlockSpec((2 * _D, _H), lambda i: (0, 0)),
            pl.BlockSpec((_D, _H), lambda i: (0, 0)),
            pl.BlockSpec((1, _H), lambda i: (0, 0)),
            pl.BlockSpec((_H, _E), lambda i: (0, 0)),
            pl.BlockSpec((1, _E), lambda i: (0, 0)),
        ],
        out_specs=pl.BlockSpec((_BLK, _E), lambda i: (i, 0)),
        out_shape=jax.ShapeDtypeStruct((_B, _E), jnp.float32),
    )(gathered, aid3, action_table, w1sn, w1a, b1r, W2, b2r)


def _remap(i):
    return ((i >> _PSH) << _PSH) | (2 * (i & (_PCOLS // 2 - 1))
                                    + ((i >> (_PSH - 1)) & 1))


def kernel(state_ids, next_state_ids, action_ids, state_table, action_table,
           W1, b1, W2, b2):
    sid = _remap(state_ids.astype(jnp.int32)).reshape(_B // _CHUNK, _CHUNK)
    nid = _remap(next_state_ids.astype(jnp.int32)).reshape(_B // _CHUNK, _CHUNK)
    table2 = _pack_table(state_table.T)
    flat = table2.reshape(2 * _PV, _D)
    gathered = _sc_gather_packed(flat, sid, nid)
    aid3 = action_ids.astype(jnp.int32).reshape(_NB, 1, _BLK)
    return _mlp(gathered, aid3, action_table, W1[:2 * _D], W1[2 * _D:],
                b1.reshape(1, _H), W2, b2.reshape(1, _E))


# speedup vs baseline: 2.3127x; 1.0983x over previous
import functools

import jax
import jax.numpy as jnp
from jax import lax
from jax.experimental import pallas as pl
from jax.experimental.pallas import tpu as pltpu
from jax.experimental.pallas import tpu_sc as plsc

_B = 16384
_V = 100000
_A = 16
_D = 64
_H = 128
_E = 64

_PCOLS = 16384
_PSH = 14
_NPB = (_V + _PCOLS - 1) // _PCOLS
_PV = _NPB * (_PCOLS // 2)

_NC = 2
_NS = 16
_NW = _NC * _NS
_RPW = _B // _NW
_CHUNK = 128
_NCHUNK = _RPW // _CHUNK


def _pack_body(tt_ref, out_ref):
    eye = (lax.broadcasted_iota(jnp.int32, (_D, _D), 0)
           == lax.broadcasted_iota(jnp.int32, (_D, _D), 1)).astype(jnp.float32)
    t = jnp.dot(tt_ref[...].T, eye, preferred_element_type=jnp.float32)
    out_ref[...] = jnp.concatenate(
        [t[:_PCOLS // 2], t[_PCOLS // 2:]], axis=1)


def _pack_table(tt):
    return pl.pallas_call(
        _pack_body,
        grid=(_NPB,),
        in_specs=[pl.BlockSpec((_D, _PCOLS), lambda i: (0, i))],
        out_specs=pl.BlockSpec((_PCOLS // 2, 2 * _D), lambda i: (i, 0)),
        out_shape=jax.ShapeDtypeStruct((_PV, 2 * _D), jnp.float32),
        compiler_params=pltpu.CompilerParams(fuse_transposed_lhs_in_matmul=True),
    )(tt)


def _sc_gather_packed(table, sid2d, nid2d):
    mesh = plsc.VectorSubcoreMesh(core_axis_name="c", subcore_axis_name="s")

    @functools.partial(
        pl.kernel,
        mesh=mesh,
        out_type=jax.ShapeDtypeStruct((_B, 2 * _D), jnp.float32),
        scratch_types=[
            pltpu.VMEM((_NCHUNK, _CHUNK), jnp.int32),
            pltpu.VMEM((_NCHUNK, _CHUNK), jnp.int32),
            pltpu.VMEM((_RPW, _D), jnp.float32),
            pltpu.VMEM((_RPW, _D), jnp.float32),
            pltpu.SemaphoreType.DMA,
        ],
        compiler_params=pltpu.CompilerParams(use_tc_tiling_on_sc=False),
    )
    def gather_kernel(table_hbm, sid_hbm, nid_hbm, out_hbm, sidx_v, nidx_v,
                      srows_v, nrows_v, sem):
        wid = lax.axis_index("s") * _NC + lax.axis_index("c")
        pltpu.sync_copy(sid_hbm.at[pl.ds(wid * _NCHUNK, _NCHUNK)], sidx_v)
        pltpu.sync_copy(nid_hbm.at[pl.ds(wid * _NCHUNK, _NCHUNK)], nidx_v)
        copies = []
        for j in range(_NCHUNK):
            copies.append(pltpu.async_copy(
                table_hbm.at[sidx_v.at[j]],
                srows_v.at[pl.ds(j * _CHUNK, _CHUNK)],
                sem,
            ))
            copies.append(pltpu.async_copy(
                table_hbm.at[nidx_v.at[j]],
                nrows_v.at[pl.ds(j * _CHUNK, _CHUNK)],
                sem,
            ))
        for c in copies:
            c.wait()
        pltpu.sync_copy(
            srows_v, out_hbm.at[pl.ds(wid * _RPW, _RPW), pl.ds(0, _D)])
        pltpu.sync_copy(
            nrows_v, out_hbm.at[pl.ds(wid * _RPW, _RPW), pl.ds(_D, _D)])

    return gather_kernel(table, sid2d, nid2d)


_BLK = 1024
_NB = _B // _BLK


def _mlp_body(g_ref, aid_ref, at_ref, w1sn_ref, w1a_ref, b1_ref, w2_ref,
              b2_ref, out_ref):
    aid = aid_ref[0, 0, :]
    onehot = (aid[:, None] == lax.broadcasted_iota(jnp.int32, (_BLK, _A), 1)
              ).astype(jnp.float32)
    aw = jnp.dot(at_ref[...], w1a_ref[...], preferred_element_type=jnp.float32)
    acc = jnp.dot(g_ref[...], w1sn_ref[...], preferred_element_type=jnp.float32)
    acc = acc + jnp.dot(onehot, aw, preferred_element_type=jnp.float32)
    h = jnp.maximum(acc + b1_ref[...], 0.0)
    out_ref[...] = (jnp.dot(h, w2_ref[...], preferred_element_type=jnp.float32)
                    + b2_ref[...]).T


def _mlp(gathered, aid3, action_table, w1sn, w1a, b1r, W2, b2r):
    return pl.pallas_call(
        _mlp_body,
        grid=(_NB,),
        in_specs=[
            pl.BlockSpec((_BLK, 2 * _D), lambda i: (i, 0)),
            pl.BlockSpec((1, 1, _BLK), lambda i: (i, 0, 0)),
            pl.BlockSpec((_A, _D), lambda i: (0, 0)),
            pl.BlockSpec((2 * _D, _H), lambda i: (0, 0)),
            pl.BlockSpec((_D, _H), lambda i: (0, 0)),
            pl.BlockSpec((1, _H), lambda i: (0, 0)),
            pl.BlockSpec((_H, _E), lambda i: (0, 0)),
            pl.BlockSpec((1, _E), lambda i: (0, 0)),
        ],
        out_specs=pl.BlockSpec((_E, _BLK), lambda i: (0, i)),
        out_shape=jax.ShapeDtypeStruct((_E, _B), jnp.float32),
    )(gathered, aid3, action_table, w1sn, w1a, b1r, W2, b2r)


def _remap(i):
    return ((i >> _PSH) << _PSH) | (2 * (i & (_PCOLS // 2 - 1))
                                    + ((i >> (_PSH - 1)) & 1))


def kernel(state_ids, next_state_ids, action_ids, state_table, action_table,
           W1, b1, W2, b2):
    sid = _remap(state_ids.astype(jnp.int32)).reshape(_B // _CHUNK, _CHUNK)
    nid = _remap(next_state_ids.astype(jnp.int32)).reshape(_B // _CHUNK, _CHUNK)
    table2 = _pack_table(state_table.T)
    flat = table2.reshape(2 * _PV, _D)
    gathered = _sc_gather_packed(flat, sid, nid)
    aid3 = action_ids.astype(jnp.int32).reshape(_NB, 1, _BLK)
    return _mlp(gathered, aid3, action_table, W1[:2 * _D], W1[2 * _D:],
                b1.reshape(1, _H), W2, b2.reshape(1, _E)).T


# speedup vs baseline: 2.5087x; 1.0847x over previous
import functools

import jax
import jax.numpy as jnp
from jax import lax
from jax.experimental import pallas as pl
from jax.experimental.pallas import tpu as pltpu
from jax.experimental.pallas import tpu_sc as plsc

_B = 16384
_V = 100000
_A = 16
_D = 64
_H = 128
_E = 64

_PCOLS = 16384
_PSH = 14
_NPB = (_V + _PCOLS - 1) // _PCOLS
_PV = _NPB * (_PCOLS // 2)

_NC = 2
_NS = 16
_NW = _NC * _NS
_RPW = _B // _NW
_CHUNK = 128
_NCHUNK = _RPW // _CHUNK


def _pack_body(tt_ref, out_ref):
    eye = (lax.broadcasted_iota(jnp.int32, (_D, _D), 0)
           == lax.broadcasted_iota(jnp.int32, (_D, _D), 1)).astype(jnp.float32)
    t = jnp.dot(tt_ref[...].T, eye, preferred_element_type=jnp.float32)
    out_ref[...] = jnp.concatenate(
        [t[:_PCOLS // 2], t[_PCOLS // 2:]], axis=1)


def _pack_table(tt):
    return pl.pallas_call(
        _pack_body,
        grid=(_NPB,),
        in_specs=[pl.BlockSpec((_D, _PCOLS), lambda i: (0, i))],
        out_specs=pl.BlockSpec((_PCOLS // 2, 2 * _D), lambda i: (i, 0)),
        out_shape=jax.ShapeDtypeStruct((_PV, 2 * _D), jnp.float32),
        compiler_params=pltpu.CompilerParams(fuse_transposed_lhs_in_matmul=True),
    )(tt)


def _sc_gather_packed(table, sid2d, nid2d):
    mesh = plsc.VectorSubcoreMesh(core_axis_name="c", subcore_axis_name="s")

    @functools.partial(
        pl.kernel,
        mesh=mesh,
        out_type=jax.ShapeDtypeStruct((_B, 2 * _D), jnp.float32),
        scratch_types=[
            pltpu.VMEM((_NCHUNK, _CHUNK), jnp.int32),
            pltpu.VMEM((_NCHUNK, _CHUNK), jnp.int32),
            pltpu.VMEM((_RPW, _D), jnp.float32),
            pltpu.VMEM((_RPW, _D), jnp.float32),
            pltpu.SemaphoreType.DMA,
        ],
        compiler_params=pltpu.CompilerParams(use_tc_tiling_on_sc=False),
    )
    def gather_kernel(table_hbm, sid_hbm, nid_hbm, out_hbm, sidx_v, nidx_v,
                      srows_v, nrows_v, sem):
        wid = lax.axis_index("s") * _NC + lax.axis_index("c")
        pltpu.sync_copy(sid_hbm.at[pl.ds(wid * _NCHUNK, _NCHUNK)], sidx_v)
        pltpu.sync_copy(nid_hbm.at[pl.ds(wid * _NCHUNK, _NCHUNK)], nidx_v)
        copies = []
        for j in range(_NCHUNK):
            copies.append(pltpu.async_copy(
                table_hbm.at[sidx_v.at[j]],
                srows_v.at[pl.ds(j * _CHUNK, _CHUNK)],
                sem,
            ))
            copies.append(pltpu.async_copy(
                table_hbm.at[nidx_v.at[j]],
                nrows_v.at[pl.ds(j * _CHUNK, _CHUNK)],
                sem,
            ))
        for c in copies:
            c.wait()
        pltpu.sync_copy(
            srows_v, out_hbm.at[pl.ds(wid * _RPW, _RPW), pl.ds(0, _D)])
        pltpu.sync_copy(
            nrows_v, out_hbm.at[pl.ds(wid * _RPW, _RPW), pl.ds(_D, _D)])

    return gather_kernel(table, sid2d, nid2d)


_BLK = 2048
_NB = _B // _BLK


def _mlp_body(g_ref, aid_ref, at_ref, w1sn_ref, w1a_ref, b1_ref, w2_ref,
              b2_ref, out_ref):
    aid = aid_ref[0, 0, :]
    onehot = (aid[:, None] == lax.broadcasted_iota(jnp.int32, (_BLK, _A), 1)
              ).astype(jnp.float32)
    aw = jnp.dot(at_ref[...], w1a_ref[...], preferred_element_type=jnp.float32)
    acc = jnp.dot(g_ref[...], w1sn_ref[...], preferred_element_type=jnp.float32)
    acc = acc + jnp.dot(onehot, aw, preferred_element_type=jnp.float32)
    h = jnp.maximum(acc + b1_ref[...], 0.0)
    out_ref[...] = (jnp.dot(h, w2_ref[...], preferred_element_type=jnp.float32)
                    + b2_ref[...]).T


def _mlp(gathered, aid3, action_table, w1sn, w1a, b1r, W2, b2r):
    return pl.pallas_call(
        _mlp_body,
        grid=(_NB,),
        in_specs=[
            pl.BlockSpec((_BLK, 2 * _D), lambda i: (i, 0)),
            pl.BlockSpec((1, 1, _BLK), lambda i: (i, 0, 0)),
            pl.BlockSpec((_A, _D), lambda i: (0, 0)),
            pl.BlockSpec((2 * _D, _H), lambda i: (0, 0)),
            pl.BlockSpec((_D, _H), lambda i: (0, 0)),
            pl.BlockSpec((1, _H), lambda i: (0, 0)),
            pl.BlockSpec((_H, _E), lambda i: (0, 0)),
            pl.BlockSpec((1, _E), lambda i: (0, 0)),
        ],
        out_specs=pl.BlockSpec((_E, _BLK), lambda i: (0, i)),
        out_shape=jax.ShapeDtypeStruct((_E, _B), jnp.float32),
    )(gathered, aid3, action_table, w1sn, w1a, b1r, W2, b2r)


def _remap(i):
    return ((i >> _PSH) << _PSH) | (2 * (i & (_PCOLS // 2 - 1))
                                    + ((i >> (_PSH - 1)) & 1))


def kernel(state_ids, next_state_ids, action_ids, state_table, action_table,
           W1, b1, W2, b2):
    sid = _remap(state_ids.astype(jnp.int32)).reshape(_B // _CHUNK, _CHUNK)
    nid = _remap(next_state_ids.astype(jnp.int32)).reshape(_B // _CHUNK, _CHUNK)
    table2 = _pack_table(state_table.T)
    flat = table2.reshape(2 * _PV, _D)
    gathered = _sc_gather_packed(flat, sid, nid)
    aid3 = action_ids.astype(jnp.int32).reshape(_NB, 1, _BLK)
    return _mlp(gathered, aid3, action_table, W1[:2 * _D], W1[2 * _D:],
                b1.reshape(1, _H), W2, b2.reshape(1, _E)).T
